# Initial kernel scaffold; baseline (speedup 1.0000x reference)
import jax
import jax.numpy as jnp
from jax.experimental import pallas as pl


def kernel(_input, sos, multiA):
    raise NotImplementedError("write your pallas kernel here")


# speedup vs baseline: 56.4720x; 56.4720x over previous
import dataclasses
import functools

import jax
import jax.numpy as jnp
from jax import lax
from jax.experimental import pallas as pl
from jax.experimental.pallas import tpu as pltpu
from jax.experimental.pallas import tpu_sc as plsc

_ROW = 64
_COL = 64
_K = 20
_BATCH = 16384
_NC = 2
_NS = 16
_NW = _NC * _NS
_CHUNK = _BATCH // _NW
_LANES = 16
_TAB = _ROW * _COL


def _table_body(sos_ref, ma_ref, tab_ref, tab0_ref):
    x = sos_ref[...]
    sp = jnp.maximum(x, 0.0) + jnp.log1p(jnp.exp(-jnp.abs(x)))
    mat = lax.dot_general(
        sp, sp, (((1,), (1,)), ((), ())),
        preferred_element_type=jnp.float32,
        precision=lax.Precision.HIGHEST,
    )
    inv = 1.0 / jnp.sqrt(jnp.sum(mat * mat))
    tab = mat * inv
    tab_ref[...] = tab
    tab0_ref[...] = tab * ma_ref[0, 0]


def _make_tables(sos, multiA):
    return pl.pallas_call(
        _table_body,
        out_shape=(
            jax.ShapeDtypeStruct((_ROW, _COL), jnp.float32),
            jax.ShapeDtypeStruct((_ROW, _COL), jnp.float32),
        ),
    )(sos.reshape(_ROW, _COL), multiA.reshape(1, 1))


def _sc_body(idx_hbm, tab_hbm, tab0_hbm, out_hbm, idx_v, tab_v, tab0_v, out_v, sem):
    wid = lax.axis_index("s") * _NC + lax.axis_index("c")
    base = wid * _CHUNK
    pltpu.sync_copy(tab_hbm, tab_v)
    pltpu.sync_copy(tab0_hbm, tab0_v)
    pltpu.sync_copy(idx_hbm.at[pl.ds(base * _K, _CHUNK * _K)], idx_v)

    lane = lax.iota(jnp.int32, _LANES)

    @pl.loop(0, _CHUNK, step=_LANES)
    def _(r):
        offs = (r + lane) * _K
        i0 = plsc.load_gather(idx_v, [offs])
        acc = plsc.load_gather(tab0_v, [i0])
        for k in range(1, _K):
            ik = plsc.load_gather(idx_v, [offs + k])
            acc = acc * plsc.load_gather(tab_v, [ik])
        out_v[pl.ds(r, _LANES)] = acc

    pltpu.sync_copy(out_v, out_hbm.at[pl.ds(base, _CHUNK)])


_SC_PARAMS = pltpu.CompilerParams()
if "needs_layout_passes" in pltpu.CompilerParams.__dataclass_fields__:
    _SC_PARAMS = dataclasses.replace(_SC_PARAMS, needs_layout_passes=False)


@functools.partial(
    pl.kernel,
    out_type=jax.ShapeDtypeStruct((_BATCH,), jnp.float32),
    compiler_params=_SC_PARAMS,
    mesh=plsc.VectorSubcoreMesh(core_axis_name="c", subcore_axis_name="s"),
    scratch_types=[
        pltpu.VMEM((_CHUNK * _K,), jnp.int32),
        pltpu.VMEM((_TAB,), jnp.float32),
        pltpu.VMEM((_TAB,), jnp.float32),
        pltpu.VMEM((_CHUNK,), jnp.float32),
        pltpu.SemaphoreType.DMA,
    ],
)
def _sc_kernel(*refs):
    _sc_body(*refs)


def kernel(_input, sos, multiA):
    idx = _input.astype(jnp.int32).reshape(-1)
    tab, tab0 = _make_tables(sos, multiA)
    return _sc_kernel(idx, tab.reshape(-1), tab0.reshape(-1))


# speedup vs baseline: 61.0719x; 1.0815x over previous
import dataclasses
import functools

import jax
import jax.numpy as jnp
from jax import lax
from jax.experimental import pallas as pl
from jax.experimental.pallas import tpu as pltpu
from jax.experimental.pallas import tpu_sc as plsc

_ROW = 64
_COL = 64
_K = 20
_BATCH = 16384
_NC = 2
_NS = 16
_NW = _NC * _NS
_CHUNK = _BATCH // _NW
_LANES = 16
_TAB = _ROW * _COL


def _table_body(sos_ref, ma_ref, tab_ref, tab0_ref):
    x = sos_ref[...]
    sp = jnp.maximum(x, 0.0) + jnp.log1p(jnp.exp(-jnp.abs(x)))
    mat = lax.dot_general(
        sp, sp, (((1,), (1,)), ((), ())),
        preferred_element_type=jnp.float32,
        precision=lax.Precision.HIGHEST,
    )
    inv = 1.0 / jnp.sqrt(jnp.sum(mat * mat))
    tab = mat * inv
    tab_ref[...] = tab
    tab0_ref[...] = tab * ma_ref[0, 0]


def _make_tables(sos, multiA):
    return pl.pallas_call(
        _table_body,
        out_shape=(
            jax.ShapeDtypeStruct((_ROW, _COL), jnp.float32),
            jax.ShapeDtypeStruct((_ROW, _COL), jnp.float32),
        ),
    )(sos.reshape(_ROW, _COL), multiA.reshape(1, 1))


def _sc_body(idx_hbm, tab_hbm, tab0_hbm, out_hbm, idx_v, tab_v, tab0_v, out_v, sem):
    wid = lax.axis_index("s") * _NC + lax.axis_index("c")
    base = wid * _CHUNK
    pltpu.sync_copy(tab_hbm, tab_v)
    pltpu.sync_copy(tab0_hbm, tab0_v)
    pltpu.sync_copy(idx_hbm.at[pl.ds(base, _CHUNK)], idx_v)

    lane = lax.iota(jnp.int32, _LANES)

    @pl.loop(0, _CHUNK, step=_LANES)
    def _(r):
        rows = r + lane
        i0 = plsc.load_gather(idx_v, [rows, jnp.zeros((_LANES,), jnp.int32)])
        acc = plsc.load_gather(tab0_v, [i0])
        for k in range(1, _K):
            ik = plsc.load_gather(idx_v, [rows, jnp.full((_LANES,), k, jnp.int32)])
            acc = acc * plsc.load_gather(tab_v, [ik])
        out_v[pl.ds(r, _LANES)] = acc

    pltpu.sync_copy(out_v, out_hbm.at[pl.ds(base, _CHUNK)])


_SC_PARAMS = pltpu.CompilerParams()
if "needs_layout_passes" in pltpu.CompilerParams.__dataclass_fields__:
    _SC_PARAMS = dataclasses.replace(_SC_PARAMS, needs_layout_passes=False)


@functools.partial(
    pl.kernel,
    out_type=jax.ShapeDtypeStruct((_BATCH,), jnp.float32),
    compiler_params=_SC_PARAMS,
    mesh=plsc.VectorSubcoreMesh(core_axis_name="c", subcore_axis_name="s"),
    scratch_types=[
        pltpu.VMEM((_CHUNK, _K), jnp.int32),
        pltpu.VMEM((_TAB,), jnp.float32),
        pltpu.VMEM((_TAB,), jnp.float32),
        pltpu.VMEM((_CHUNK,), jnp.float32),
        pltpu.SemaphoreType.DMA,
    ],
)
def _sc_kernel(*refs):
    _sc_body(*refs)


def kernel(_input, sos, multiA):
    tab, tab0 = _make_tables(sos, multiA)
    return _sc_kernel(_input, tab.reshape(-1), tab0.reshape(-1))
